# speedup vs baseline: 1.9957x; 1.9957x over previous
import functools

import jax
import jax.numpy as jnp
from jax import lax
from jax.experimental import pallas as pl
from jax.experimental.pallas import tpu as pltpu
from jax.experimental.pallas import tpu_sc as plsc

MAX_LEN = 8192
D = 1024
B_TOTAL = 4 * 8192

_info = plsc.get_sparse_core_info()
NC = _info.num_cores
NS = _info.num_subcores
NW = NC * NS
B_PER_W = B_TOTAL // NW
CH = 32
N_CHUNKS = B_PER_W // CH


@jax.jit
def _gather_rows(table, idx3):
  mesh = plsc.VectorSubcoreMesh(core_axis_name="c", subcore_axis_name="s")

  @functools.partial(
      pl.kernel,
      mesh=mesh,
      out_type=jax.ShapeDtypeStruct((B_TOTAL, D), jnp.float32),
      scratch_types=[
          pltpu.VMEM((N_CHUNKS, CH), jnp.int32),
          pltpu.VMEM((CH, D), jnp.float32),
          pltpu.SemaphoreType.DMA,
      ],
  )
  def k(table_hbm, idx_hbm, out_hbm, idx_v, rows_v, sem):
    wid = lax.axis_index("s") * NC + lax.axis_index("c")
    base = wid * B_PER_W
    pltpu.sync_copy(idx_hbm.at[wid], idx_v)

    def body(c, carry):
      pltpu.async_copy(table_hbm.at[idx_v.at[c]], rows_v, sem).wait()
      pltpu.sync_copy(rows_v, out_hbm.at[pl.ds(base + c * CH, CH)])
      return carry

    lax.fori_loop(0, N_CHUNKS, body, 0, unroll=False)

  return k(table, idx3)


def kernel(inputs, positions, position_embeddings):
  idx3 = positions.reshape(NW, N_CHUNKS, CH).astype(jnp.int32)
  out = _gather_rows(position_embeddings, idx3)
  return out.reshape(inputs.shape)


# speedup vs baseline: 2.3764x; 1.1908x over previous
import functools

import jax
import jax.numpy as jnp
from jax import lax
from jax.experimental import pallas as pl
from jax.experimental.pallas import tpu as pltpu
from jax.experimental.pallas import tpu_sc as plsc

MAX_LEN = 8192
D = 1024
B_TOTAL = 4 * 8192

_info = plsc.get_sparse_core_info()
NC = _info.num_cores
NS = _info.num_subcores
NW = NC * NS
B_PER_W = B_TOTAL // NW
CH = 16
N_CHUNKS = B_PER_W // CH
NB = 4


@jax.jit
def _gather_rows(table, idx3):
  mesh = plsc.VectorSubcoreMesh(core_axis_name="c", subcore_axis_name="s")

  @functools.partial(
      pl.kernel,
      mesh=mesh,
      out_type=jax.ShapeDtypeStruct((B_TOTAL, D), jnp.float32),
      scratch_types=[
          pltpu.VMEM((N_CHUNKS, CH), jnp.int32),
          pltpu.VMEM((NB, CH, D), jnp.float32),
          pltpu.SemaphoreType.DMA((NB,)),
          pltpu.SemaphoreType.DMA((NB,)),
      ],
  )
  def k(table_hbm, idx_hbm, out_hbm, idx_v, rows_v, gsem, ssem):
    wid = lax.axis_index("s") * NC + lax.axis_index("c")
    base = wid * B_PER_W
    pltpu.sync_copy(idx_hbm.at[wid], idx_v)

    def start_gather(c, b):
      pltpu.async_copy(table_hbm.at[idx_v.at[c]], rows_v.at[b], gsem.at[b])

    def wait_gather(c, b):
      pltpu.make_async_copy(
          table_hbm.at[idx_v.at[c]], rows_v.at[b], gsem.at[b]).wait()

    def start_scatter(c, b):
      pltpu.async_copy(
          rows_v.at[b], out_hbm.at[pl.ds(base + c * CH, CH)], ssem.at[b])

    def wait_scatter(c, b):
      pltpu.make_async_copy(
          rows_v.at[b], out_hbm.at[pl.ds(base + c * CH, CH)], ssem.at[b]).wait()

    for b in range(NB - 1):
      start_gather(b, b)

    def outer(i, carry):
      g = i * NB
      for b in range(NB):
        c = g + b
        bp = (b + NB - 1) % NB
        @pl.when(c >= 1)
        def _():
          wait_scatter(c - 1, bp)
        @pl.when(c + NB - 1 < N_CHUNKS)
        def _():
          start_gather(c + NB - 1, bp)
        wait_gather(c, b)
        start_scatter(c, b)
      return carry

    lax.fori_loop(0, N_CHUNKS // NB, outer, 0, unroll=False)

    wait_scatter(N_CHUNKS - 1, (N_CHUNKS - 1) % NB)

  return k(table, idx3)


def kernel(inputs, positions, position_embeddings):
  idx3 = positions.reshape(NW, N_CHUNKS, CH).astype(jnp.int32)
  out = _gather_rows(position_embeddings, idx3)
  return out.reshape(inputs.shape)


# speedup vs baseline: 2.3874x; 1.0046x over previous
import functools

import jax
import jax.numpy as jnp
from jax import lax
from jax.experimental import pallas as pl
from jax.experimental.pallas import tpu as pltpu
from jax.experimental.pallas import tpu_sc as plsc

MAX_LEN = 8192
D = 1024
B_TOTAL = 4 * 8192

_info = plsc.get_sparse_core_info()
NC = _info.num_cores
NS = _info.num_subcores
NW = NC * NS
B_PER_W = B_TOTAL // NW
CH = 32
N_CHUNKS = B_PER_W // CH
NB = 3
N_MAIN = (N_CHUNKS // NB) * NB


@jax.jit
def _gather_rows(table, idx3):
  mesh = plsc.VectorSubcoreMesh(core_axis_name="c", subcore_axis_name="s")

  @functools.partial(
      pl.kernel,
      mesh=mesh,
      out_type=jax.ShapeDtypeStruct((B_TOTAL, D), jnp.float32),
      scratch_types=[
          pltpu.VMEM((N_CHUNKS, CH), jnp.int32),
          pltpu.VMEM((NB, CH, D), jnp.float32),
          pltpu.SemaphoreType.DMA((NB,)),
          pltpu.SemaphoreType.DMA((NB,)),
      ],
  )
  def k(table_hbm, idx_hbm, out_hbm, idx_v, rows_v, gsem, ssem):
    wid = lax.axis_index("s") * NC + lax.axis_index("c")
    base = wid * B_PER_W
    pltpu.sync_copy(idx_hbm.at[wid], idx_v)

    def start_gather(c, b):
      pltpu.async_copy(table_hbm.at[idx_v.at[c]], rows_v.at[b], gsem.at[b])

    def wait_gather(c, b):
      pltpu.make_async_copy(
          table_hbm.at[idx_v.at[c]], rows_v.at[b], gsem.at[b]).wait()

    def start_scatter(c, b):
      pltpu.async_copy(
          rows_v.at[b], out_hbm.at[pl.ds(base + c * CH, CH)], ssem.at[b])

    def wait_scatter(c, b):
      pltpu.make_async_copy(
          rows_v.at[b], out_hbm.at[pl.ds(base + c * CH, CH)], ssem.at[b]).wait()

    def step(c, b, static):
      bp = (b + NB - 1) % NB
      if static:
        if c >= 1:
          wait_scatter(c - 1, bp)
        if c + NB - 1 < N_CHUNKS:
          start_gather(c + NB - 1, bp)
      else:
        @pl.when(c >= 1)
        def _():
          wait_scatter(c - 1, bp)
        @pl.when(c + NB - 1 < N_CHUNKS)
        def _():
          start_gather(c + NB - 1, bp)
      wait_gather(c, b)
      start_scatter(c, b)

    for b in range(NB - 1):
      start_gather(b, b)

    def outer(i, carry):
      g = i * NB
      for b in range(NB):
        step(g + b, b, static=False)
      return carry

    lax.fori_loop(0, N_MAIN // NB, outer, 0, unroll=False)

    for c in range(N_MAIN, N_CHUNKS):
      step(c, c % NB, static=True)

    wait_scatter(N_CHUNKS - 1, (N_CHUNKS - 1) % NB)

  return k(table, idx3)


def kernel(inputs, positions, position_embeddings):
  idx3 = positions.reshape(NW, N_CHUNKS, CH).astype(jnp.int32)
  out = _gather_rows(position_embeddings, idx3)
  return out.reshape(inputs.shape)
